# baseline (speedup 1.0000x reference)
import functools

import jax
import jax.numpy as jnp
from jax import lax
from jax.experimental import pallas as pl
from jax.experimental.pallas import tpu as pltpu
from jax.experimental.pallas import tpu_sc as plsc

VOCAB = 100000
EMB = 64
HID = 128
N_THEME = 10
N_SENTI = 3
B = 1024
L = 200
BL = B * L

NC = 2
NSUB = 16
NW = NC * NSUB
TOK_PER_W = BL // NW
CHUNK = 128
N_CHUNKS = TOK_PER_W // CHUNK

BB = 8


def _sc_gather(token_flat, emb_table):
    mesh = plsc.VectorSubcoreMesh(core_axis_name="c", subcore_axis_name="s")

    @functools.partial(
        pl.kernel,
        out_type=jax.ShapeDtypeStruct((BL, EMB), jnp.float32),
        mesh=mesh,
        scratch_types=[
            pltpu.VMEM((CHUNK,), jnp.int32),
            pltpu.VMEM((CHUNK, EMB), jnp.float32),
            pltpu.SemaphoreType.DMA,
        ],
        compiler_params=pltpu.CompilerParams(use_tc_tiling_on_sc=False),
    )
    def gather_kernel(tok_hbm, table_hbm, out_hbm, idx_v, rows_v, sem):
        wid = lax.axis_index("s") * NC + lax.axis_index("c")
        base = wid * TOK_PER_W

        def body(c, carry):
            off = base + c * CHUNK
            pltpu.sync_copy(tok_hbm.at[pl.ds(off, CHUNK)], idx_v)
            pltpu.async_copy(table_hbm.at[idx_v], rows_v, sem).wait()
            pltpu.sync_copy(rows_v, out_hbm.at[pl.ds(off, CHUNK)])
            return carry

        lax.fori_loop(0, N_CHUNKS, body, 0)

    return gather_kernel(token_flat, emb_table)


def _tc_body(len_ref, emb_ref, mt_ref, ms_ref, wt_ref, kt_ref, ws_ref,
             ks_ref, ut_ref, bt_ref, us_ref, bs_ref, out_t_ref, out_s_ref):
    embx = emb_ref[...]
    zt = (jnp.dot(embx, wt_ref[...], preferred_element_type=jnp.float32)
          + jnp.dot(mt_ref[...], kt_ref[...], preferred_element_type=jnp.float32))
    zs = (jnp.dot(embx, ws_ref[...], preferred_element_type=jnp.float32)
          + jnp.dot(ms_ref[...], ks_ref[...], preferred_element_type=jnp.float32))
    ht = jnp.tanh(zt)
    hs = jnp.tanh(zs)

    lens = len_ref[...]
    rows = lax.broadcasted_iota(jnp.int32, (BB, BB * L), 0)
    cols = lax.broadcasted_iota(jnp.int32, (BB, BB * L), 1)
    q = cols // L
    r = cols - q * L
    valid = (q == rows) & (r < lens)
    sel = jnp.where(valid, 1.0, 0.0)

    denom = jnp.maximum(lens.astype(jnp.float32), 1.0)
    pooled_t = jnp.dot(sel, ht, preferred_element_type=jnp.float32) / denom
    pooled_s = jnp.dot(sel, hs, preferred_element_type=jnp.float32) / denom
    out_t_ref[...] = (jnp.dot(pooled_t, ut_ref[...],
                              preferred_element_type=jnp.float32) + bt_ref[...])
    out_s_ref[...] = (jnp.dot(pooled_s, us_ref[...],
                              preferred_element_type=jnp.float32) + bs_ref[...])


def _tc_forward(len2, emb_flat, mt2, ms2, W_theme, K_theme, W_senti, K_senti,
                U_theme, bt2, U_senti, bs2):
    grid = (B // BB,)
    full = lambda shape: pl.BlockSpec(shape, lambda i: (0, 0))
    return pl.pallas_call(
        _tc_body,
        grid=grid,
        in_specs=[
            pl.BlockSpec((BB, 1), lambda i: (i, 0)),
            pl.BlockSpec((BB * L, EMB), lambda i: (i, 0)),
            pl.BlockSpec((BB * L, N_THEME), lambda i: (i, 0)),
            pl.BlockSpec((BB * L, N_SENTI), lambda i: (i, 0)),
            full((EMB, HID)),
            full((N_THEME, HID)),
            full((EMB, HID)),
            full((N_SENTI, HID)),
            full((HID, N_THEME)),
            full((1, N_THEME)),
            full((HID, N_SENTI)),
            full((1, N_SENTI)),
        ],
        out_specs=(
            pl.BlockSpec((BB, N_THEME), lambda i: (i, 0)),
            pl.BlockSpec((BB, N_SENTI), lambda i: (i, 0)),
        ),
        out_shape=(
            jax.ShapeDtypeStruct((B, N_THEME), jnp.float32),
            jax.ShapeDtypeStruct((B, N_SENTI), jnp.float32),
        ),
    )(len2, emb_flat, mt2, ms2, W_theme, K_theme, W_senti, K_senti,
      U_theme, bt2, U_senti, bs2)


def kernel(token, token_len, mention_theme, mention_senti, emb_table,
           W_theme, K_theme, W_senti, K_senti,
           U_theme, b_theme, U_senti, b_senti):
    token_flat = token.reshape(BL)
    emb_flat = _sc_gather(token_flat, emb_table)
    len2 = token_len.reshape(B, 1)
    mt2 = mention_theme.reshape(BL, N_THEME)
    ms2 = mention_senti.reshape(BL, N_SENTI)
    bt2 = b_theme.reshape(1, N_THEME)
    bs2 = b_senti.reshape(1, N_SENTI)
    return _tc_forward(len2, emb_flat, mt2, ms2, W_theme, K_theme,
                       W_senti, K_senti, U_theme, bt2, U_senti, bs2)


# speedup vs baseline: 1.1017x; 1.1017x over previous
import functools

import jax
import jax.numpy as jnp
from jax import lax
from jax.experimental import pallas as pl
from jax.experimental.pallas import tpu as pltpu
from jax.experimental.pallas import tpu_sc as plsc

VOCAB = 100000
EMB = 64
HID = 128
N_THEME = 10
N_SENTI = 3
B = 1024
L = 200
BL = B * L

NC = 2
NSUB = 16
NW = NC * NSUB
DIMS_PER_W = EMB // NW
CH = 12800
N_CH = BL // CH
GRP = CH // 16


def _sc_gather_t(tok_flat, table_t):
    mesh = plsc.VectorSubcoreMesh(core_axis_name="c", subcore_axis_name="s")

    @functools.partial(
        pl.kernel,
        out_type=jax.ShapeDtypeStruct((EMB, BL), jnp.float32),
        mesh=mesh,
        scratch_types=[
            pltpu.VMEM((VOCAB,), jnp.float32),
            pltpu.VMEM((CH,), jnp.int32),
            pltpu.VMEM((CH,), jnp.float32),
        ],
        compiler_params=pltpu.CompilerParams(needs_layout_passes=False),
    )
    def gather_kernel(tok_hbm, table_hbm, out_hbm, row_v, idx_v, out_v):
        wid = lax.axis_index("s") * NC + lax.axis_index("c")

        for dd in range(DIMS_PER_W):
            d = wid * DIMS_PER_W + dd
            pltpu.sync_copy(table_hbm.at[d], row_v)

            def chunk_body(ci, carry):
                off = ci * CH
                pltpu.sync_copy(tok_hbm.at[pl.ds(off, CH)], idx_v)

                def g_body(g, c2):
                    sl = pl.ds(g * 16, 16)
                    out_v[sl] = plsc.load_gather(row_v, [idx_v[sl]])
                    return c2

                lax.fori_loop(0, GRP, g_body, 0)
                pltpu.sync_copy(out_v, out_hbm.at[d, pl.ds(off, CH)])
                return carry

            lax.fori_loop(0, N_CH, chunk_body, 0)

    return gather_kernel(tok_flat, table_t)


def _tc_body(len_ref, emb_ref, mt_ref, ms_ref, wt_ref, kt_ref, ws_ref,
             ks_ref, ut_ref, bt_ref, us_ref, bs_ref, out_t_ref, out_s_ref,
             acc_t, acc_s):
    i = pl.program_id(0)

    @pl.when(i == 0)
    def _init():
        acc_t[...] = jnp.zeros((HID, B), jnp.float32)
        acc_s[...] = jnp.zeros((HID, B), jnp.float32)

    embx = emb_ref[...]
    mtx = mt_ref[:, 0, 0, :]
    msx = ms_ref[:, 0, 0, :]
    zt = (jnp.dot(wt_ref[...], embx, preferred_element_type=jnp.float32)
          + jnp.dot(kt_ref[...], mtx, preferred_element_type=jnp.float32))
    zs = (jnp.dot(ws_ref[...], embx, preferred_element_type=jnp.float32)
          + jnp.dot(ks_ref[...], msx, preferred_element_type=jnp.float32))

    lens = len_ref[...]
    w = jnp.where(lens > i, 1.0, 0.0)
    acc_t[...] += jnp.tanh(zt) * w
    acc_s[...] += jnp.tanh(zs) * w

    @pl.when(i == L - 1)
    def _fin():
        denom = jnp.maximum(lens.astype(jnp.float32), 1.0)
        pt = acc_t[...] / denom
        ps = acc_s[...] / denom
        out_t_ref[...] = (jnp.dot(ut_ref[...], pt,
                                  preferred_element_type=jnp.float32)
                          + bt_ref[...])
        out_s_ref[...] = (jnp.dot(us_ref[...], ps,
                                  preferred_element_type=jnp.float32)
                          + bs_ref[...])


def _tc_forward(len2, emb_t, mt_t, ms_t, wt_t, kt_t, ws_t, ks_t,
                ut_t, bt2, us_t, bs2):
    full2 = lambda shape: pl.BlockSpec(shape, lambda i: (0, 0))
    return pl.pallas_call(
        _tc_body,
        grid=(L,),
        in_specs=[
            full2((1, B)),
            pl.BlockSpec((EMB, B), lambda i: (0, i)),
            pl.BlockSpec((N_THEME, 1, 1, B), lambda i: (0, i, 0, 0)),
            pl.BlockSpec((N_SENTI, 1, 1, B), lambda i: (0, i, 0, 0)),
            full2((HID, EMB)),
            full2((HID, N_THEME)),
            full2((HID, EMB)),
            full2((HID, N_SENTI)),
            full2((N_THEME, HID)),
            full2((N_THEME, 1)),
            full2((N_SENTI, HID)),
            full2((N_SENTI, 1)),
        ],
        out_specs=(
            pl.BlockSpec((N_THEME, B), lambda i: (0, 0)),
            pl.BlockSpec((N_SENTI, B), lambda i: (0, 0)),
        ),
        out_shape=(
            jax.ShapeDtypeStruct((N_THEME, B), jnp.float32),
            jax.ShapeDtypeStruct((N_SENTI, B), jnp.float32),
        ),
        scratch_shapes=[
            pltpu.VMEM((HID, B), jnp.float32),
            pltpu.VMEM((HID, B), jnp.float32),
        ],
    )(len2, emb_t, mt_t, ms_t, wt_t, kt_t, ws_t, ks_t, ut_t, bt2, us_t, bs2)


def kernel(token, token_len, mention_theme, mention_senti, emb_table,
           W_theme, K_theme, W_senti, K_senti,
           U_theme, b_theme, U_senti, b_senti):
    tok_flat = token.T.reshape(BL)
    table_t = emb_table.T
    emb_t = _sc_gather_t(tok_flat, table_t)

    len2 = token_len.reshape(1, B)
    mt_t = mention_theme.transpose(2, 1, 0).reshape(N_THEME, L, 1, B)
    ms_t = mention_senti.transpose(2, 1, 0).reshape(N_SENTI, L, 1, B)
    out_t, out_s = _tc_forward(
        len2, emb_t, mt_t, ms_t,
        W_theme.T, K_theme.T, W_senti.T, K_senti.T,
        U_theme.T, b_theme.reshape(N_THEME, 1),
        U_senti.T, b_senti.reshape(N_SENTI, 1))
    return (out_t.T, out_s.T)


# speedup vs baseline: 1.9977x; 1.8132x over previous
import functools

import jax
import jax.numpy as jnp
from jax import lax
from jax.experimental import pallas as pl
from jax.experimental.pallas import tpu as pltpu
from jax.experimental.pallas import tpu_sc as plsc

VOCAB = 100000
EMB = 64
HID = 128
N_THEME = 10
N_SENTI = 3
B = 1024
L = 200
BL = B * L

NC = 2
NSUB = 16
NW = NC * NSUB
DIMS_PER_W = EMB // NW
CH = 6400
N_CH = BL // CH
GRP = CH // 16
UNROLL = 8

LB = 8


def _sc_gather_t(tok_flat, table_t):
    mesh = plsc.VectorSubcoreMesh(core_axis_name="c", subcore_axis_name="s")

    @functools.partial(
        pl.kernel,
        out_type=jax.ShapeDtypeStruct((EMB, BL), jnp.float32),
        mesh=mesh,
        scratch_types=[
            pltpu.VMEM((VOCAB,), jnp.float32),
            pltpu.VMEM((CH,), jnp.int32),
            pltpu.VMEM((CH,), jnp.int32),
            pltpu.VMEM((CH,), jnp.float32),
            pltpu.VMEM((CH,), jnp.float32),
            pltpu.SemaphoreType.DMA,
            pltpu.SemaphoreType.DMA,
            pltpu.SemaphoreType.DMA,
            pltpu.SemaphoreType.DMA,
        ],
        compiler_params=pltpu.CompilerParams(needs_layout_passes=False),
    )
    def gather_kernel(tok_hbm, table_hbm, out_hbm, row_v,
                      idx_a, idx_b, out_a, out_b,
                      isem_a, isem_b, osem_a, osem_b):
        wid = lax.axis_index("s") * NC + lax.axis_index("c")
        idx_bufs = (idx_a, idx_b)
        out_bufs = (out_a, out_b)
        isems = (isem_a, isem_b)
        osems = (osem_a, osem_b)

        def gather_chunk(idx_v, out_v):
            def g_body(g, carry):
                base = g * (16 * UNROLL)
                for u in range(UNROLL):
                    sl = pl.ds(base + u * 16, 16)
                    out_v[sl] = plsc.load_gather(row_v, [idx_v[sl]])
                return carry
            lax.fori_loop(0, GRP // UNROLL, g_body, 0)

        for dd in range(DIMS_PER_W):
            d = wid * DIMS_PER_W + dd
            pltpu.sync_copy(table_hbm.at[d], row_v)
            pltpu.async_copy(tok_hbm.at[pl.ds(0, CH)], idx_a, isem_a)
            pltpu.async_copy(tok_hbm.at[pl.ds(CH, CH)], idx_b, isem_b)

            def chunk_pair(ci, carry):
                for p in range(2):
                    c = ci * 2 + p
                    off = c * CH
                    idx_v, out_v = idx_bufs[p], out_bufs[p]
                    isem, osem = isems[p], osems[p]
                    pltpu.make_async_copy(
                        tok_hbm.at[pl.ds(off, CH)], idx_v, isem).wait()
                    @pl.when(jnp.logical_or(c >= 2, dd > 0))
                    def _drain():
                        pltpu.make_async_copy(
                            out_v, out_hbm.at[d, pl.ds(off, CH)], osem).wait()
                    gather_chunk(idx_v, out_v)
                    pltpu.async_copy(
                        out_v, out_hbm.at[d, pl.ds(off, CH)], osem)
                    @pl.when(c + 2 < N_CH)
                    def _prefetch():
                        noff = (c + 2) * CH
                        pltpu.async_copy(
                            tok_hbm.at[pl.ds(noff, CH)], idx_v, isem)
                return carry

            lax.fori_loop(0, N_CH // 2, chunk_pair, 0)

        d_last = wid * DIMS_PER_W + DIMS_PER_W - 1
        for p in range(2):
            off = (N_CH - 2 + p) * CH
            pltpu.make_async_copy(
                out_bufs[p], out_hbm.at[d_last, pl.ds(off, CH)],
                osems[p]).wait()

    return gather_kernel(tok_flat, table_t)


def _tc_body(len_ref, emb_ref, mt_ref, ms_ref, wt_ref, kt_ref, ws_ref,
             ks_ref, ut_ref, bt_ref, us_ref, bs_ref, out_t_ref, out_s_ref,
             acc_t, acc_s):
    i = pl.program_id(0)

    @pl.when(i == 0)
    def _init():
        acc_t[...] = jnp.zeros((HID, B), jnp.float32)
        acc_s[...] = jnp.zeros((HID, B), jnp.float32)

    lens = len_ref[...]
    zt_emb = jnp.dot(wt_ref[...], emb_ref[...],
                     preferred_element_type=jnp.float32)
    zs_emb = jnp.dot(ws_ref[...], emb_ref[...],
                     preferred_element_type=jnp.float32)
    at = acc_t[...]
    as_ = acc_s[...]
    for u in range(LB):
        mtx = mt_ref[:, u, 0, :]
        msx = ms_ref[:, u, 0, :]
        zt = zt_emb[:, u * B:(u + 1) * B] + jnp.dot(
            kt_ref[...], mtx, preferred_element_type=jnp.float32)
        zs = zs_emb[:, u * B:(u + 1) * B] + jnp.dot(
            ks_ref[...], msx, preferred_element_type=jnp.float32)
        w = jnp.where(lens > i * LB + u, 1.0, 0.0)
        at = at + jnp.tanh(zt) * w
        as_ = as_ + jnp.tanh(zs) * w
    acc_t[...] = at
    acc_s[...] = as_

    @pl.when(i == (L // LB) - 1)
    def _fin():
        denom = jnp.maximum(lens.astype(jnp.float32), 1.0)
        pt = at / denom
        ps = as_ / denom
        out_t_ref[...] = (jnp.dot(ut_ref[...], pt,
                                  preferred_element_type=jnp.float32)
                          + bt_ref[...])
        out_s_ref[...] = (jnp.dot(us_ref[...], ps,
                                  preferred_element_type=jnp.float32)
                          + bs_ref[...])


def _tc_forward(len2, emb_t, mt_t, ms_t, wt_t, kt_t, ws_t, ks_t,
                ut_t, bt2, us_t, bs2):
    full2 = lambda shape: pl.BlockSpec(shape, lambda i: (0, 0))
    return pl.pallas_call(
        _tc_body,
        grid=(L // LB,),
        in_specs=[
            full2((1, B)),
            pl.BlockSpec((EMB, LB * B), lambda i: (0, i)),
            pl.BlockSpec((N_THEME, LB, 1, B), lambda i: (0, i, 0, 0)),
            pl.BlockSpec((N_SENTI, LB, 1, B), lambda i: (0, i, 0, 0)),
            full2((HID, EMB)),
            full2((HID, N_THEME)),
            full2((HID, EMB)),
            full2((HID, N_SENTI)),
            full2((N_THEME, HID)),
            full2((N_THEME, 1)),
            full2((N_SENTI, HID)),
            full2((N_SENTI, 1)),
        ],
        out_specs=(
            pl.BlockSpec((N_THEME, B), lambda i: (0, 0)),
            pl.BlockSpec((N_SENTI, B), lambda i: (0, 0)),
        ),
        out_shape=(
            jax.ShapeDtypeStruct((N_THEME, B), jnp.float32),
            jax.ShapeDtypeStruct((N_SENTI, B), jnp.float32),
        ),
        scratch_shapes=[
            pltpu.VMEM((HID, B), jnp.float32),
            pltpu.VMEM((HID, B), jnp.float32),
        ],
    )(len2, emb_t, mt_t, ms_t, wt_t, kt_t, ws_t, ks_t, ut_t, bt2, us_t, bs2)


def kernel(token, token_len, mention_theme, mention_senti, emb_table,
           W_theme, K_theme, W_senti, K_senti,
           U_theme, b_theme, U_senti, b_senti):
    tok_flat = token.T.reshape(BL)
    table_t = emb_table.T
    emb_t = _sc_gather_t(tok_flat, table_t)

    len2 = token_len.reshape(1, B)
    mt_t = mention_theme.transpose(2, 1, 0).reshape(N_THEME, L, 1, B)
    ms_t = mention_senti.transpose(2, 1, 0).reshape(N_SENTI, L, 1, B)
    out_t, out_s = _tc_forward(
        len2, emb_t, mt_t, ms_t,
        W_theme.T, K_theme.T, W_senti.T, K_senti.T,
        U_theme.T, b_theme.reshape(N_THEME, 1),
        U_senti.T, b_senti.reshape(N_SENTI, 1))
    return (out_t.T, out_s.T)


# speedup vs baseline: 2.4483x; 1.2256x over previous
import functools

import jax
import jax.numpy as jnp
from jax import lax
from jax.experimental import pallas as pl
from jax.experimental.pallas import tpu as pltpu
from jax.experimental.pallas import tpu_sc as plsc

VOCAB = 100000
EMB = 64
HID = 128
N_THEME = 10
N_SENTI = 3
B = 1024
L = 200
BL = B * L

NC = 2
NSUB = 16
NW = NC * NSUB
DIMS_PER_W = EMB // NW
CH = 6400
N_CH = BL // CH
GRP = CH // 16
UNROLL = 8

LB = 8


def _sc_gather_t(tok_flat, table_t):
    mesh = plsc.VectorSubcoreMesh(core_axis_name="c", subcore_axis_name="s")

    @functools.partial(
        pl.kernel,
        out_type=jax.ShapeDtypeStruct((EMB, BL), jnp.float32),
        mesh=mesh,
        scratch_types=[
            pltpu.VMEM((VOCAB,), jnp.float32),
            pltpu.VMEM((CH,), jnp.int32),
            pltpu.VMEM((CH,), jnp.int32),
            pltpu.VMEM((CH,), jnp.float32),
            pltpu.VMEM((CH,), jnp.float32),
            pltpu.SemaphoreType.DMA,
            pltpu.SemaphoreType.DMA,
            pltpu.SemaphoreType.DMA,
            pltpu.SemaphoreType.DMA,
        ],
        compiler_params=pltpu.CompilerParams(needs_layout_passes=False),
    )
    def gather_kernel(tok_hbm, table_hbm, out_hbm, row_v,
                      idx_a, idx_b, out_a, out_b,
                      isem_a, isem_b, osem_a, osem_b):
        wid = lax.axis_index("s") * NC + lax.axis_index("c")
        idx_bufs = (idx_a, idx_b)
        out_bufs = (out_a, out_b)
        isems = (isem_a, isem_b)
        osems = (osem_a, osem_b)

        def gather_chunk(idx_v, out_v):
            @plsc.parallel_loop(0, CH, 16, unroll=UNROLL)
            def _g(base):
                sl = pl.ds(base, 16)
                out_v[sl] = plsc.load_gather(row_v, [idx_v[sl]])

        for dd in range(DIMS_PER_W):
            d = wid * DIMS_PER_W + dd
            pltpu.sync_copy(table_hbm.at[d], row_v)
            pltpu.async_copy(tok_hbm.at[pl.ds(0, CH)], idx_a, isem_a)
            pltpu.async_copy(tok_hbm.at[pl.ds(CH, CH)], idx_b, isem_b)

            def chunk_pair(ci, carry):
                for p in range(2):
                    c = ci * 2 + p
                    off = c * CH
                    idx_v, out_v = idx_bufs[p], out_bufs[p]
                    isem, osem = isems[p], osems[p]
                    pltpu.make_async_copy(
                        tok_hbm.at[pl.ds(off, CH)], idx_v, isem).wait()
                    @pl.when(jnp.logical_or(c >= 2, dd > 0))
                    def _drain():
                        pltpu.make_async_copy(
                            out_v, out_hbm.at[d, pl.ds(off, CH)], osem).wait()
                    gather_chunk(idx_v, out_v)
                    pltpu.async_copy(
                        out_v, out_hbm.at[d, pl.ds(off, CH)], osem)
                    @pl.when(c + 2 < N_CH)
                    def _prefetch():
                        noff = (c + 2) * CH
                        pltpu.async_copy(
                            tok_hbm.at[pl.ds(noff, CH)], idx_v, isem)
                return carry

            lax.fori_loop(0, N_CH // 2, chunk_pair, 0)

        d_last = wid * DIMS_PER_W + DIMS_PER_W - 1
        for p in range(2):
            off = (N_CH - 2 + p) * CH
            pltpu.make_async_copy(
                out_bufs[p], out_hbm.at[d_last, pl.ds(off, CH)],
                osems[p]).wait()

    return gather_kernel(tok_flat, table_t)


def _tc_body(len_ref, emb_ref, mt_ref, ms_ref, wt_ref, kt_ref, ws_ref,
             ks_ref, ut_ref, bt_ref, us_ref, bs_ref, out_t_ref, out_s_ref,
             acc_t, acc_s):
    i = pl.program_id(0)

    @pl.when(i == 0)
    def _init():
        acc_t[...] = jnp.zeros((HID, B), jnp.float32)
        acc_s[...] = jnp.zeros((HID, B), jnp.float32)

    lens = len_ref[...]
    zt_emb = jnp.dot(wt_ref[...], emb_ref[...],
                     preferred_element_type=jnp.float32)
    zs_emb = jnp.dot(ws_ref[...], emb_ref[...],
                     preferred_element_type=jnp.float32)
    at = acc_t[...]
    as_ = acc_s[...]
    for u in range(LB):
        mtx = mt_ref[:, u, 0, :]
        msx = ms_ref[:, u, 0, :]
        zt = zt_emb[:, u * B:(u + 1) * B] + jnp.dot(
            kt_ref[...], mtx, preferred_element_type=jnp.float32)
        zs = zs_emb[:, u * B:(u + 1) * B] + jnp.dot(
            ks_ref[...], msx, preferred_element_type=jnp.float32)
        w = jnp.where(lens > i * LB + u, 1.0, 0.0)
        at = at + jnp.tanh(zt) * w
        as_ = as_ + jnp.tanh(zs) * w
    acc_t[...] = at
    acc_s[...] = as_

    @pl.when(i == (L // LB) - 1)
    def _fin():
        denom = jnp.maximum(lens.astype(jnp.float32), 1.0)
        pt = at / denom
        ps = as_ / denom
        out_t_ref[...] = (jnp.dot(ut_ref[...], pt,
                                  preferred_element_type=jnp.float32)
                          + bt_ref[...])
        out_s_ref[...] = (jnp.dot(us_ref[...], ps,
                                  preferred_element_type=jnp.float32)
                          + bs_ref[...])


def _tc_forward(len2, emb_t, mt_t, ms_t, wt_t, kt_t, ws_t, ks_t,
                ut_t, bt2, us_t, bs2):
    full2 = lambda shape: pl.BlockSpec(shape, lambda i: (0, 0))
    return pl.pallas_call(
        _tc_body,
        grid=(L // LB,),
        in_specs=[
            full2((1, B)),
            pl.BlockSpec((EMB, LB * B), lambda i: (0, i)),
            pl.BlockSpec((N_THEME, LB, 1, B), lambda i: (0, i, 0, 0)),
            pl.BlockSpec((N_SENTI, LB, 1, B), lambda i: (0, i, 0, 0)),
            full2((HID, EMB)),
            full2((HID, N_THEME)),
            full2((HID, EMB)),
            full2((HID, N_SENTI)),
            full2((N_THEME, HID)),
            full2((N_THEME, 1)),
            full2((N_SENTI, HID)),
            full2((N_SENTI, 1)),
        ],
        out_specs=(
            pl.BlockSpec((N_THEME, B), lambda i: (0, 0)),
            pl.BlockSpec((N_SENTI, B), lambda i: (0, 0)),
        ),
        out_shape=(
            jax.ShapeDtypeStruct((N_THEME, B), jnp.float32),
            jax.ShapeDtypeStruct((N_SENTI, B), jnp.float32),
        ),
        scratch_shapes=[
            pltpu.VMEM((HID, B), jnp.float32),
            pltpu.VMEM((HID, B), jnp.float32),
        ],
    )(len2, emb_t, mt_t, ms_t, wt_t, kt_t, ws_t, ks_t, ut_t, bt2, us_t, bs2)


def kernel(token, token_len, mention_theme, mention_senti, emb_table,
           W_theme, K_theme, W_senti, K_senti,
           U_theme, b_theme, U_senti, b_senti):
    tok_flat = token.T.reshape(BL)
    table_t = emb_table.T
    emb_t = _sc_gather_t(tok_flat, table_t)

    len2 = token_len.reshape(1, B)
    mt_t = mention_theme.transpose(2, 1, 0).reshape(N_THEME, L, 1, B)
    ms_t = mention_senti.transpose(2, 1, 0).reshape(N_SENTI, L, 1, B)
    out_t, out_s = _tc_forward(
        len2, emb_t, mt_t, ms_t,
        W_theme.T, K_theme.T, W_senti.T, K_senti.T,
        U_theme.T, b_theme.reshape(N_THEME, 1),
        U_senti.T, b_senti.reshape(N_SENTI, 1))
    return (out_t.T, out_s.T)
